# Initial kernel scaffold; baseline (speedup 1.0000x reference)
import jax
import jax.numpy as jnp
from jax.experimental import pallas as pl


def kernel(x, source, k, Wq, bq, Wk, bk, Wv, bv, Wm, bm):
    raise NotImplementedError("write your pallas kernel here")


# speedup vs baseline: 18.9062x; 18.9062x over previous
import functools

import jax
import jax.numpy as jnp
from jax.experimental import pallas as pl
from jax.experimental.pallas import tpu as pltpu

D_MODEL = 1024
NUM_HEADS = 16
HEAD_DIM = 64
SEQ = 2048
TOPK = 128
INT_MIN = -2147483648

BQ = 256


def _proj_kernel(x_ref, src_ref, wq_ref, wkv_ref, bq_ref, bkv_ref,
                 q_ref, kv_ref):
    x = x_ref[...]
    s = src_ref[...]
    q_ref[...] = jax.lax.dot_general(
        wq_ref[...], x, (((1,), (0,)), ((), ())),
        preferred_element_type=jnp.float32) + bq_ref[...]
    kv_ref[...] = jax.lax.dot_general(
        wkv_ref[...], s, (((1,), (0,)), ((), ())),
        preferred_element_type=jnp.float32) + bkv_ref[...]


def _attn_kernel(q_ref, k_ref, v_ref, o_ref):
    qh = q_ref[0]
    kh = k_ref[0]
    s = jax.lax.dot_general(
        qh, kh, (((0,), (0,)), ((), ())),
        preferred_element_type=jnp.float32) * (1.0 / 8.0)

    b = jax.lax.bitcast_convert_type(s, jnp.int32)
    key = jnp.where(b < 0, b ^ 0x7FFFFFFF, b)

    c = jnp.sum((key >= 0).astype(jnp.int32), axis=1, keepdims=True)
    pos_side = c >= TOPK
    arr = jnp.where(pos_side, key, key ^ INT_MIN)
    rank = jnp.where(pos_side, TOPK, TOPK - c)

    def body(i, prefix):
        cand = prefix | (1 << (30 - i))
        cnt = jnp.sum((arr >= cand).astype(jnp.int32), axis=1, keepdims=True)
        return jnp.where(cnt >= rank, cand, prefix)

    prefix = jax.lax.fori_loop(
        0, 31, body, jnp.zeros((s.shape[0], 1), jnp.int32))
    tkey = jnp.where(pos_side, prefix, prefix ^ INT_MIN)

    mask = key >= tkey
    m = jnp.max(s, axis=1, keepdims=True)
    p = jnp.where(mask, jnp.exp(s - m), 0.0)
    p = p / jnp.sum(p, axis=1, keepdims=True)

    o_ref[0] = jax.lax.dot_general(
        v_ref[0], p, (((1,), (1,)), ((), ())),
        preferred_element_type=jnp.float32)


def _out_kernel(wm_ref, o_ref, bm_ref, y_ref):
    y_ref[...] = jax.lax.dot_general(
        wm_ref[...], o_ref[...], (((1,), (0,)), ((), ())),
        preferred_element_type=jnp.float32) + bm_ref[...]


def kernel(x, source, k, Wq, bq, Wk, bk, Wv, bv, Wm, bm):
    del k
    x2 = x[0]
    src2 = source[0]

    def rperm(W):
        return W.reshape(HEAD_DIM, NUM_HEADS, D_MODEL).transpose(1, 0, 2) \
                .reshape(D_MODEL, D_MODEL)

    def bperm(bvec):
        return bvec.reshape(HEAD_DIM, NUM_HEADS).T.reshape(D_MODEL, 1)

    Wq_p = rperm(Wq)
    Wkv_p = jnp.concatenate([rperm(Wk), rperm(Wv)], axis=0)
    bq_p = bperm(bq)
    bkv_p = jnp.concatenate([bperm(bk), bperm(bv)], axis=0)
    Wm_p = Wm.reshape(D_MODEL, HEAD_DIM, NUM_HEADS).transpose(0, 2, 1) \
             .reshape(D_MODEL, D_MODEL)

    q, kv = pl.pallas_call(
        _proj_kernel,
        out_shape=(
            jax.ShapeDtypeStruct((D_MODEL, SEQ), jnp.float32),
            jax.ShapeDtypeStruct((2 * D_MODEL, SEQ), jnp.float32),
        ),
    )(x2, src2, Wq_p, Wkv_p, bq_p, bkv_p)

    q3 = q.reshape(NUM_HEADS, HEAD_DIM, SEQ)
    k3 = kv[:D_MODEL].reshape(NUM_HEADS, HEAD_DIM, SEQ)
    v3 = kv[D_MODEL:].reshape(NUM_HEADS, HEAD_DIM, SEQ)

    nq = SEQ // BQ
    o3 = pl.pallas_call(
        _attn_kernel,
        grid=(NUM_HEADS, nq),
        in_specs=[
            pl.BlockSpec((1, HEAD_DIM, BQ), lambda h, qi: (h, 0, qi)),
            pl.BlockSpec((1, HEAD_DIM, SEQ), lambda h, qi: (h, 0, 0)),
            pl.BlockSpec((1, HEAD_DIM, SEQ), lambda h, qi: (h, 0, 0)),
        ],
        out_specs=pl.BlockSpec((1, HEAD_DIM, BQ), lambda h, qi: (h, 0, qi)),
        out_shape=jax.ShapeDtypeStruct((NUM_HEADS, HEAD_DIM, SEQ),
                                       jnp.float32),
    )(q3, k3, v3)

    o2 = o3.reshape(D_MODEL, SEQ)
    y = pl.pallas_call(
        _out_kernel,
        out_shape=jax.ShapeDtypeStruct((D_MODEL, SEQ), jnp.float32),
    )(Wm_p, o2, bm.reshape(D_MODEL, 1))
    return y[None]
